# Initial kernel scaffold; baseline (speedup 1.0000x reference)
import jax
import jax.numpy as jnp
from jax.experimental import pallas as pl


def kernel(x, edge_index, edge_type, conv_weight, conv_root, conv_bias, lin_W, lin_b):
    raise NotImplementedError("write your pallas kernel here")


# speedup vs baseline: 6.4782x; 6.4782x over previous
import functools

import jax
import jax.numpy as jnp
from jax import lax
from jax.experimental import pallas as pl
from jax.experimental.pallas import tpu as pltpu
from jax.experimental.pallas import tpu_sc as plsc

N = 10000
F = 128
H = 128
R = 8
C = 16
E = 320000

NT = 16
GROUP = 128
GPT = 8 * -(-E // (NT * GROUP * 8))
E_PAD = NT * GROUP * GPT
LHALF = 5120
LROWS = 5248
ROWS_PER_TILE = LROWS // NT
N_CNT = 10240
NCROWS = N_CNT // GROUP
BN = 1000


def _phase1(x, conv_weight):
    def body(x_ref, w_ref, o_ref):
        o_ref[0] = jnp.dot(x_ref[...], w_ref[0],
                           preferred_element_type=jnp.float32)

    return pl.pallas_call(
        body,
        grid=(R, N // BN),
        in_specs=[
            pl.BlockSpec((BN, F), lambda r, b: (b, 0)),
            pl.BlockSpec((1, F, H), lambda r, b: (r, 0, 0)),
        ],
        out_specs=pl.BlockSpec((1, BN, H), lambda r, b: (r, b, 0)),
        out_shape=jax.ShapeDtypeStruct((R, N, H), jnp.float32),
    )(x, conv_weight)


def _phase2(h_flat, src2d, et2d, dst2d):
    mesh = plsc.VectorSubcoreMesh(core_axis_name="c", subcore_axis_name="s")

    @functools.partial(
        pl.kernel,
        out_type=(
            jax.ShapeDtypeStruct((2, LROWS, H), jnp.float32),
            jax.ShapeDtypeStruct((2, NCROWS, GROUP), jnp.float32),
        ),
        mesh=mesh,
        scratch_types=[
            pltpu.VMEM((GPT, GROUP), jnp.int32),
            pltpu.VMEM((GPT, GROUP), jnp.int32),
            pltpu.VMEM((GPT, GROUP), jnp.int32),
            pltpu.VMEM((GROUP, H), jnp.float32),
            pltpu.VMEM((NCROWS, GROUP), jnp.float32),
            pltpu.VMEM((NCROWS,), jnp.int32),
            pltpu.VMEM_SHARED((LROWS, H), jnp.float32),
            pltpu.VMEM_SHARED((NCROWS, GROUP), jnp.float32),
            pltpu.SemaphoreType.DMA,
        ],
        compiler_params=pltpu.CompilerParams(needs_layout_passes=False),
    )
    def k(h_hbm, src_hbm, et_hbm, dst_hbm, agg_out, cnt_out,
          src_v, dst_v, idx_v, rows_v, cnt_v, idxc, agg_sh, cnt_sh, sem):
        cid = lax.axis_index("c")
        sid = lax.axis_index("s")
        gbase = sid * GPT

        pltpu.sync_copy(src_hbm.at[pl.ds(gbase, GPT)], src_v)
        pltpu.sync_copy(et_hbm.at[pl.ds(gbase, GPT)], idx_v)
        pltpu.sync_copy(dst_hbm.at[pl.ds(gbase, GPT)], dst_v)

        zero16 = jnp.zeros((16,), jnp.float32)
        iota16 = lax.iota(jnp.int32, 16)

        def zrow(r, carry):
            for c in range(H // 16):
                rows_v[r, pl.ds(c * 16, 16)] = zero16
            return carry

        lax.fori_loop(0, GROUP, zrow, 0)

        def zcnt(r, carry):
            for c in range(GROUP // 16):
                cnt_v[r, pl.ds(c * 16, 16)] = zero16
            return carry

        lax.fori_loop(0, NCROWS, zcnt, 0)
        for t in range(NCROWS // 16):
            idxc[pl.ds(t * 16, 16)] = iota16 + (t * 16)

        rowbase = sid * ROWS_PER_TILE
        pltpu.sync_copy(rows_v, agg_sh.at[pl.ds(rowbase, GROUP)])
        pltpu.sync_copy(rows_v, agg_sh.at[pl.ds(rowbase + GROUP, GROUP)])
        pltpu.sync_copy(rows_v.at[pl.ds(0, ROWS_PER_TILE - 2 * GROUP)],
                        agg_sh.at[pl.ds(rowbase + 2 * GROUP,
                                        ROWS_PER_TILE - 2 * GROUP)])

        @pl.when(sid == 0)
        def _():
            pltpu.sync_copy(cnt_v, cnt_sh)

        plsc.subcore_barrier()

        lo = cid * LHALF

        def gbody(g, carry):
            for j in range(GROUP // 16):
                s16 = src_v[g, pl.ds(j * 16, 16)]
                e16 = idx_v[g, pl.ds(j * 16, 16)]
                idx_v[g, pl.ds(j * 16, 16)] = e16 * N + s16
            cp = pltpu.async_copy(h_hbm.at[idx_v.at[g]], rows_v, sem)
            for j in range(GROUP // 16):
                d16 = dst_v[g, pl.ds(j * 16, 16)]
                cnts, last = plsc.scan_count(d16)
                row = lax.shift_right_logical(d16, 7)
                col = lax.bitwise_and(d16, GROUP - 1)
                plsc.addupdate_scatter(cnt_v, [row, col],
                                       cnts.astype(jnp.float32), mask=last)
                local = d16 - lo
                inrange = (local >= 0) & (local < LHALF)
                spread = LHALF + lax.bitwise_and(d16, GROUP - 1)
                dst_v[g, pl.ds(j * 16, 16)] = jnp.where(inrange, local, spread)
            cp.wait()
            pltpu.sync_copy(rows_v, agg_sh.at[dst_v.at[g]], add=True)
            return carry

        lax.fori_loop(0, GPT, gbody, 0)

        pltpu.sync_copy(cnt_v, cnt_sh.at[idxc], add=True)
        plsc.subcore_barrier()

        pltpu.sync_copy(agg_sh.at[pl.ds(rowbase, ROWS_PER_TILE)],
                        agg_out.at[cid, pl.ds(rowbase, ROWS_PER_TILE)])

        @pl.when(sid == 0)
        def _():
            pltpu.sync_copy(cnt_sh, cnt_out.at[cid])

    return k(h_flat, src2d, et2d, dst2d)


def _phase3(acc2, cnt_col, x, conv_root, conv_bias, lin_W, lin_b):
    def body(a_ref, c_ref, x_ref, root_ref, bias_ref, lw_ref, lb_ref, o_ref):
        cnt = c_ref[...]
        agg = a_ref[...] / jnp.maximum(cnt, 1.0)
        out1 = agg + jnp.dot(x_ref[...], root_ref[...],
                             preferred_element_type=jnp.float32) + bias_ref[...]
        out1 = jnp.maximum(out1, 0.0)
        o_ref[...] = jnp.dot(out1, lw_ref[...],
                             preferred_element_type=jnp.float32) + lb_ref[...]

    return pl.pallas_call(
        body,
        grid=(N // BN,),
        in_specs=[
            pl.BlockSpec((BN, H), lambda b: (b, 0)),
            pl.BlockSpec((BN, 1), lambda b: (b, 0)),
            pl.BlockSpec((BN, F), lambda b: (b, 0)),
            pl.BlockSpec((F, H), lambda b: (0, 0)),
            pl.BlockSpec((1, H), lambda b: (0, 0)),
            pl.BlockSpec((H, C), lambda b: (0, 0)),
            pl.BlockSpec((1, C), lambda b: (0, 0)),
        ],
        out_specs=pl.BlockSpec((BN, C), lambda b: (b, 0)),
        out_shape=jax.ShapeDtypeStruct((N, C), jnp.float32),
    )(acc2, cnt_col, x, conv_root, conv_bias, lin_W, lin_b)


def kernel(x, edge_index, edge_type, conv_weight, conv_root, conv_bias, lin_W, lin_b):
    h = _phase1(x, conv_weight)
    h_flat = h.reshape(R * N, H)

    src = edge_index[0]
    dst = edge_index[1]
    pad = E_PAD - E
    src2d = jnp.concatenate([src, jnp.zeros((pad,), jnp.int32)]).reshape(-1, GROUP)
    et2d = jnp.concatenate([edge_type, jnp.zeros((pad,), jnp.int32)]).reshape(-1, GROUP)
    dst2d = jnp.concatenate([dst, jnp.full((pad,), N, jnp.int32)]).reshape(-1, GROUP)

    acc, cnt_planes = _phase2(h_flat, src2d, et2d, dst2d)
    acc2 = acc[:, :LHALF].reshape(2 * LHALF, H)
    cnt_col = cnt_planes[0].reshape(N_CNT, 1)
    return _phase3(acc2, cnt_col, x, conv_root, conv_bias.reshape(1, H),
                   lin_W, lin_b.reshape(1, C))
